# baseline (speedup 1.0000x reference)
import functools

import jax
import jax.numpy as jnp
from jax import lax
from jax.experimental import pallas as pl
from jax.experimental.pallas import tpu as pltpu
from jax.experimental.pallas import tpu_sc as plsc

N_SPINS = 20
BATCH = 16384
NUM_WORKERS = 32
B_PER_W = BATCH // NUM_WORKERS
ROWS = 4


def _tc_bitpack(sig_ref, idx_ref):
    s = sig_ref[...]
    pw = jnp.int32(1) << lax.broadcasted_iota(jnp.int32, (N_SPINS, 1), 0)
    bits = jnp.where(s > 0.0, pw, jnp.int32(0))
    idx_ref[...] = jnp.sum(bits, axis=0)


def _sc_gather(idx_hbm, amps_hbm, out_hbm, idx_v, out_v, gsem):
    nc = 2
    wid = lax.axis_index("s") * nc + lax.axis_index("c")
    base = wid * B_PER_W

    pltpu.sync_copy(idx_hbm.at[pl.ds(base, B_PER_W)], idx_v)

    gathers = [
        pltpu.async_copy(amps_hbm.at[idx_v.at[pl.ds(r * 128, 128)]],
                         out_v.at[pl.ds(r * 128, 128)], gsem)
        for r in range(ROWS)
    ]
    for cp in gathers:
        cp.wait()

    pltpu.sync_copy(out_v, out_hbm.at[pl.ds(base, B_PER_W)])


@jax.jit
def kernel(sigma, amps):
    sig_t = sigma.T
    blk = 2048
    idx = pl.pallas_call(
        _tc_bitpack,
        grid=(BATCH // blk,),
        in_specs=[pl.BlockSpec((N_SPINS, blk), lambda j: (0, j))],
        out_specs=pl.BlockSpec((blk,), lambda j: (j,)),
        out_shape=jax.ShapeDtypeStruct((BATCH,), jnp.int32),
    )(sig_t)

    mesh = plsc.VectorSubcoreMesh(core_axis_name="c", subcore_axis_name="s")
    k = functools.partial(
        pl.kernel,
        mesh=mesh,
        out_type=jax.ShapeDtypeStruct((BATCH,), jnp.float32),
        scratch_types=[
            pltpu.VMEM((B_PER_W,), jnp.int32),
            pltpu.VMEM((B_PER_W,), jnp.float32),
            pltpu.SemaphoreType.DMA,
        ],
        compiler_params=pltpu.CompilerParams(needs_layout_passes=False),
    )(_sc_gather)
    return k(idx, amps)


# speedup vs baseline: 1.1350x; 1.1350x over previous
import functools

import jax
import jax.numpy as jnp
from jax import lax
from jax.experimental import pallas as pl
from jax.experimental.pallas import tpu as pltpu
from jax.experimental.pallas import tpu_sc as plsc

N_SPINS = 20
BATCH = 16384
NUM_WORKERS = 32
B_PER_W = BATCH // NUM_WORKERS
ROWS = 4


def _tc_bitpack(sig_ref, idx_ref):
    s = sig_ref[...]
    pw = jnp.int32(1) << lax.broadcasted_iota(jnp.int32, (N_SPINS, 1), 0)
    bits = jnp.where(s > 0.0, pw, jnp.int32(0))
    idx_ref[...] = jnp.sum(bits, axis=0)


def _sc_gather(idx_hbm, amps_hbm, out_hbm, idx_v, out_v, gsem):
    nc = 2
    wid = lax.axis_index("s") * nc + lax.axis_index("c")
    base = wid * B_PER_W

    pltpu.sync_copy(idx_hbm.at[pl.ds(base, B_PER_W)], idx_v)

    gathers = [
        pltpu.async_copy(amps_hbm.at[idx_v.at[pl.ds(r * 128, 128)]],
                         out_v.at[pl.ds(r * 128, 128)], gsem)
        for r in range(ROWS)
    ]
    for cp in gathers:
        cp.wait()

    pltpu.sync_copy(out_v, out_hbm.at[pl.ds(base, B_PER_W)])


@jax.jit
def kernel(sigma, amps):
    sig_t = sigma.T
    blk = BATCH // 2
    idx = pl.pallas_call(
        _tc_bitpack,
        grid=(BATCH // blk,),
        in_specs=[pl.BlockSpec((N_SPINS, blk), lambda j: (0, j))],
        out_specs=pl.BlockSpec((blk,), lambda j: (j,)),
        out_shape=jax.ShapeDtypeStruct((BATCH,), jnp.int32),
    )(sig_t)

    mesh = plsc.VectorSubcoreMesh(core_axis_name="c", subcore_axis_name="s")
    k = functools.partial(
        pl.kernel,
        mesh=mesh,
        out_type=jax.ShapeDtypeStruct((BATCH,), jnp.float32),
        scratch_types=[
            pltpu.VMEM((B_PER_W,), jnp.int32),
            pltpu.VMEM((B_PER_W,), jnp.float32),
            pltpu.SemaphoreType.DMA,
        ],
        compiler_params=pltpu.CompilerParams(needs_layout_passes=False),
    )(_sc_gather)
    return k(idx, amps)
